# Initial kernel scaffold; baseline (speedup 1.0000x reference)
import jax
import jax.numpy as jnp
from jax.experimental import pallas as pl


def kernel(x, W_gate, W_experts, b_experts):
    raise NotImplementedError("write your pallas kernel here")


# speedup vs baseline: 2.3912x; 2.3912x over previous
import functools

import jax
import jax.numpy as jnp
from jax import lax
from jax.experimental import pallas as pl
from jax.experimental.pallas import tpu as pltpu

E = 8
D = 1024
BT = 512


def _moe_kernel(y_ref, wg_ref, we_ref, be_ref, out_ref, z_ref, aux_ref, p_scr):
    i = pl.program_id(0)
    j = pl.program_id(1)

    @pl.when(j == 0)
    def _router():
        logits = lax.dot_general(
            y_ref[...], wg_ref[...], (((1,), (1,)), ((), ())),
            preferred_element_type=jnp.float32)
        m = jnp.max(logits, axis=-1, keepdims=True)
        ex = jnp.exp(logits - m)
        s = jnp.sum(ex, axis=-1, keepdims=True)
        p = ex / s
        p_scr[...] = p / jnp.sum(p, axis=-1, keepdims=True)
        lse = m[:, 0] + jnp.log(s[:, 0])
        z_part = jnp.sum(lse * lse)
        p_part = jnp.sum(p)

        @pl.when(i == 0)
        def _init():
            z_ref[0, 0] = 0.0
            aux_ref[0, 0] = 0.0

        z_ref[0, 0] += z_part
        aux_ref[0, 0] += p_part

    w = we_ref[0]
    mm = lax.dot_general(
        y_ref[...], w, (((1,), (1,)), ((), ())),
        preferred_element_type=jnp.float32)
    onehot = (lax.broadcasted_iota(jnp.int32, (1, E), 1) == j).astype(jnp.float32)
    pe = jnp.sum(p_scr[...] * onehot, axis=1, keepdims=True)
    acc = pe * (mm + be_ref[0])

    @pl.when(j == 0)
    def _first():
        out_ref[...] = acc

    @pl.when(j != 0)
    def _rest():
        out_ref[...] += acc


@jax.jit
def kernel(x, W_gate, W_experts, b_experts):
    bs, seq, d = x.shape
    y = x.reshape(-1, d)
    T = y.shape[0]
    nt = T // BT

    out, z, aux = pl.pallas_call(
        _moe_kernel,
        grid=(nt, E),
        in_specs=[
            pl.BlockSpec((BT, D), lambda i, j: (i, 0)),
            pl.BlockSpec((E, D), lambda i, j: (0, 0)),
            pl.BlockSpec((1, D, D), lambda i, j: (j, 0, 0)),
            pl.BlockSpec((1, 1, D), lambda i, j: (j, 0, 0)),
        ],
        out_specs=[
            pl.BlockSpec((BT, D), lambda i, j: (i, 0)),
            pl.BlockSpec(memory_space=pltpu.SMEM),
            pl.BlockSpec(memory_space=pltpu.SMEM),
        ],
        out_shape=[
            jax.ShapeDtypeStruct((T, D), jnp.float32),
            jax.ShapeDtypeStruct((1, 1), jnp.float32),
            jax.ShapeDtypeStruct((1, 1), jnp.float32),
        ],
        scratch_shapes=[pltpu.VMEM((BT, E), jnp.float32)],
    )(y, W_gate, W_experts, b_experts.reshape(E, 1, D))

    z_loss = z[0, 0] / T
    aux_loss = aux[0, 0] * (E / T)
    return out.reshape(bs, seq, d), z_loss, aux_loss


# speedup vs baseline: 2.5675x; 1.0737x over previous
import functools

import jax
import jax.numpy as jnp
from jax import lax
from jax.experimental import pallas as pl
from jax.experimental.pallas import tpu as pltpu

E = 8
D = 1024
BT = 512


def _moe_kernel(y_ref, wg_ref, we_ref, be_ref, out_ref, z_ref, aux_ref, p_scr, yb_scr):
    i = pl.program_id(0)
    j = pl.program_id(1)

    @pl.when(j == 0)
    def _router():
        yb_scr[...] = y_ref[...].astype(jnp.bfloat16)
        logits = lax.dot_general(
            y_ref[...], wg_ref[...], (((1,), (1,)), ((), ())),
            preferred_element_type=jnp.float32)
        m = jnp.max(logits, axis=-1, keepdims=True)
        ex = jnp.exp(logits - m)
        s = jnp.sum(ex, axis=-1, keepdims=True)
        p = ex / s
        p_scr[...] = p / jnp.sum(p, axis=-1, keepdims=True)
        lse = m[:, 0] + jnp.log(s[:, 0])
        z_part = jnp.sum(lse * lse)
        p_part = jnp.sum(p)

        @pl.when(i == 0)
        def _init():
            z_ref[0, 0] = 0.0
            aux_ref[0, 0] = 0.0

        z_ref[0, 0] += z_part
        aux_ref[0, 0] += p_part

    w = we_ref[0]
    mm = lax.dot_general(
        yb_scr[...], w, (((1,), (1,)), ((), ())),
        preferred_element_type=jnp.float32)
    onehot = (lax.broadcasted_iota(jnp.int32, (1, E), 1) == j).astype(jnp.float32)
    pe = jnp.sum(p_scr[...] * onehot, axis=1, keepdims=True)
    acc = pe * (mm + be_ref[0])

    @pl.when(j == 0)
    def _first():
        out_ref[...] = acc

    @pl.when(j != 0)
    def _rest():
        out_ref[...] += acc


@jax.jit
def kernel(x, W_gate, W_experts, b_experts):
    bs, seq, d = x.shape
    y = x.reshape(-1, d)
    T = y.shape[0]
    nt = T // BT

    out, z, aux = pl.pallas_call(
        _moe_kernel,
        grid=(nt, E),
        in_specs=[
            pl.BlockSpec((BT, D), lambda i, j: (i, 0)),
            pl.BlockSpec((E, D), lambda i, j: (0, 0)),
            pl.BlockSpec((1, D, D), lambda i, j: (j, 0, 0)),
            pl.BlockSpec((1, 1, D), lambda i, j: (j, 0, 0)),
        ],
        out_specs=[
            pl.BlockSpec((BT, D), lambda i, j: (i, 0)),
            pl.BlockSpec(memory_space=pltpu.SMEM),
            pl.BlockSpec(memory_space=pltpu.SMEM),
        ],
        out_shape=[
            jax.ShapeDtypeStruct((T, D), jnp.float32),
            jax.ShapeDtypeStruct((1, 1), jnp.float32),
            jax.ShapeDtypeStruct((1, 1), jnp.float32),
        ],
        scratch_shapes=[
            pltpu.VMEM((BT, E), jnp.float32),
            pltpu.VMEM((BT, D), jnp.bfloat16),
        ],
    )(y, W_gate, W_experts.astype(jnp.bfloat16), b_experts.reshape(E, 1, D))

    z_loss = z[0, 0] / T
    aux_loss = aux[0, 0] * (E / T)
    return out.reshape(bs, seq, d), z_loss, aux_loss


# speedup vs baseline: 2.9293x; 1.1409x over previous
import jax
import jax.numpy as jnp
from jax import lax
from jax.experimental import pallas as pl
from jax.experimental.pallas import tpu as pltpu

E = 8
D = 1024
BT = 512


def _moe_kernel(y_ref, wg_ref, wt_ref, be_ref, out_ref, z_ref, aux_ref, ycat_scr):
    i = pl.program_id(0)

    y = y_ref[...]
    logits = lax.dot_general(
        y, wg_ref[...], (((1,), (1,)), ((), ())),
        preferred_element_type=jnp.float32)
    m = jnp.max(logits, axis=-1, keepdims=True)
    ex = jnp.exp(logits - m)
    s = jnp.sum(ex, axis=-1, keepdims=True)
    p = ex / s
    pn = p / jnp.sum(p, axis=-1, keepdims=True)
    lse = m[:, 0] + jnp.log(s[:, 0])
    z_part = jnp.sum(lse * lse)
    p_part = jnp.sum(p)

    @pl.when(i == 0)
    def _init():
        z_ref[0, 0] = 0.0
        aux_ref[0, 0] = 0.0

    z_ref[0, 0] += z_part
    aux_ref[0, 0] += p_part

    for e in range(E):
        ycat_scr[:, e * D:(e + 1) * D] = (pn[:, e:e + 1] * y).astype(jnp.bfloat16)

    mm = lax.dot_general(
        ycat_scr[...], wt_ref[...], (((1,), (0,)), ((), ())),
        preferred_element_type=jnp.float32)
    bias = lax.dot_general(
        pn, be_ref[...], (((1,), (0,)), ((), ())),
        preferred_element_type=jnp.float32)
    out_ref[...] = mm + bias


@jax.jit
def kernel(x, W_gate, W_experts, b_experts):
    bs, seq, d = x.shape
    y = x.reshape(-1, d)
    T = y.shape[0]
    nt = T // BT

    Wt = W_experts.transpose(0, 2, 1).reshape(E * D, D).astype(jnp.bfloat16)

    out, z, aux = pl.pallas_call(
        _moe_kernel,
        grid=(nt,),
        in_specs=[
            pl.BlockSpec((BT, D), lambda i: (i, 0)),
            pl.BlockSpec((E, D), lambda i: (0, 0)),
            pl.BlockSpec((E * D, D), lambda i: (0, 0)),
            pl.BlockSpec((E, D), lambda i: (0, 0)),
        ],
        out_specs=[
            pl.BlockSpec((BT, D), lambda i: (i, 0)),
            pl.BlockSpec(memory_space=pltpu.SMEM),
            pl.BlockSpec(memory_space=pltpu.SMEM),
        ],
        out_shape=[
            jax.ShapeDtypeStruct((T, D), jnp.float32),
            jax.ShapeDtypeStruct((1, 1), jnp.float32),
            jax.ShapeDtypeStruct((1, 1), jnp.float32),
        ],
        scratch_shapes=[pltpu.VMEM((BT, E * D), jnp.bfloat16)],
    )(y, W_gate, Wt, b_experts)

    z_loss = z[0, 0] / T
    aux_loss = aux[0, 0] * (E / T)
    return out.reshape(bs, seq, d), z_loss, aux_loss


# speedup vs baseline: 2.9627x; 1.0114x over previous
import jax
import jax.numpy as jnp
from jax import lax
from jax.experimental import pallas as pl
from jax.experimental.pallas import tpu as pltpu

E = 8
D = 1024
BT = 512


def _moe_kernel(y_ref, wg_ref, wt_ref, be_ref, out_ref, z_ref, aux_ref, ycat_scr):
    i = pl.program_id(0)

    y = y_ref[...]
    logits = lax.dot_general(
        y, wg_ref[...], (((1,), (1,)), ((), ())),
        preferred_element_type=jnp.float32)
    m = jnp.max(logits, axis=-1, keepdims=True)
    ex = jnp.exp(logits - m)
    s = jnp.sum(ex, axis=-1, keepdims=True)
    p = ex / s
    pn = p / jnp.sum(p, axis=-1, keepdims=True)
    lse = m[:, 0] + jnp.log(s[:, 0])
    z_part = jnp.sum(lse * lse)
    p_part = jnp.sum(p)

    @pl.when(i == 0)
    def _init():
        z_ref[0, 0] = 0.0
        aux_ref[0, 0] = 0.0

    z_ref[0, 0] += z_part
    aux_ref[0, 0] += p_part

    for e in range(E):
        ycat_scr[:, e * D:(e + 1) * D] = (pn[:, e:e + 1] * y).astype(jnp.bfloat16)

    mm = lax.dot_general(
        ycat_scr[...], wt_ref[...], (((1,), (1,)), ((), ())),
        preferred_element_type=jnp.float32)
    bias = lax.dot_general(
        pn, be_ref[...], (((1,), (0,)), ((), ())),
        preferred_element_type=jnp.float32)
    out_ref[...] = mm + bias


@jax.jit
def kernel(x, W_gate, W_experts, b_experts):
    bs, seq, d = x.shape
    y = x.reshape(-1, d)
    T = y.shape[0]
    nt = T // BT

    Wt = W_experts.transpose(1, 0, 2).reshape(D, E * D).astype(jnp.bfloat16)

    out, z, aux = pl.pallas_call(
        _moe_kernel,
        grid=(nt,),
        in_specs=[
            pl.BlockSpec((BT, D), lambda i: (i, 0)),
            pl.BlockSpec((E, D), lambda i: (0, 0)),
            pl.BlockSpec((D, E * D), lambda i: (0, 0)),
            pl.BlockSpec((E, D), lambda i: (0, 0)),
        ],
        out_specs=[
            pl.BlockSpec((BT, D), lambda i: (i, 0)),
            pl.BlockSpec(memory_space=pltpu.SMEM),
            pl.BlockSpec(memory_space=pltpu.SMEM),
        ],
        out_shape=[
            jax.ShapeDtypeStruct((T, D), jnp.float32),
            jax.ShapeDtypeStruct((1, 1), jnp.float32),
            jax.ShapeDtypeStruct((1, 1), jnp.float32),
        ],
        scratch_shapes=[pltpu.VMEM((BT, E * D), jnp.bfloat16)],
    )(y, W_gate, Wt, b_experts)

    z_loss = z[0, 0] / T
    aux_loss = aux[0, 0] * (E / T)
    return out.reshape(bs, seq, d), z_loss, aux_loss


# speedup vs baseline: 2.9891x; 1.0089x over previous
import jax
import jax.numpy as jnp
from jax import lax
from jax.experimental import pallas as pl
from jax.experimental.pallas import tpu as pltpu

E = 8
D = 1024
BT = 1024


def _moe_kernel(y_ref, wg_ref, wt_ref, be_ref, out_ref, z_ref, aux_ref, ycat_scr):
    i = pl.program_id(0)

    y = y_ref[...]
    logits = lax.dot_general(
        y, wg_ref[...], (((1,), (1,)), ((), ())),
        preferred_element_type=jnp.float32)
    m = jnp.max(logits, axis=-1, keepdims=True)
    ex = jnp.exp(logits - m)
    s = jnp.sum(ex, axis=-1, keepdims=True)
    p = ex / s
    pn = p / jnp.sum(p, axis=-1, keepdims=True)
    lse = m[:, 0] + jnp.log(s[:, 0])
    z_part = jnp.sum(lse * lse)
    p_part = jnp.sum(p)

    @pl.when(i == 0)
    def _init():
        z_ref[0, 0] = 0.0
        aux_ref[0, 0] = 0.0

    z_ref[0, 0] += z_part
    aux_ref[0, 0] += p_part

    for e in range(E):
        ycat_scr[:, e * D:(e + 1) * D] = (pn[:, e:e + 1] * y).astype(jnp.bfloat16)

    mm = lax.dot_general(
        ycat_scr[...], wt_ref[...], (((1,), (1,)), ((), ())),
        preferred_element_type=jnp.float32)
    bias = lax.dot_general(
        pn, be_ref[...], (((1,), (0,)), ((), ())),
        preferred_element_type=jnp.float32)
    out_ref[...] = mm + bias


@jax.jit
def kernel(x, W_gate, W_experts, b_experts):
    bs, seq, d = x.shape
    y = x.reshape(-1, d)
    T = y.shape[0]
    nt = T // BT

    Wt = W_experts.transpose(1, 0, 2).reshape(D, E * D).astype(jnp.bfloat16)

    out, z, aux = pl.pallas_call(
        _moe_kernel,
        grid=(nt,),
        in_specs=[
            pl.BlockSpec((BT, D), lambda i: (i, 0)),
            pl.BlockSpec((E, D), lambda i: (0, 0)),
            pl.BlockSpec((D, E * D), lambda i: (0, 0)),
            pl.BlockSpec((E, D), lambda i: (0, 0)),
        ],
        out_specs=[
            pl.BlockSpec((BT, D), lambda i: (i, 0)),
            pl.BlockSpec(memory_space=pltpu.SMEM),
            pl.BlockSpec(memory_space=pltpu.SMEM),
        ],
        out_shape=[
            jax.ShapeDtypeStruct((T, D), jnp.float32),
            jax.ShapeDtypeStruct((1, 1), jnp.float32),
            jax.ShapeDtypeStruct((1, 1), jnp.float32),
        ],
        scratch_shapes=[pltpu.VMEM((BT, E * D), jnp.bfloat16)],
    )(y, W_gate, Wt, b_experts)

    z_loss = z[0, 0] / T
    aux_loss = aux[0, 0] * (E / T)
    return out.reshape(bs, seq, d), z_loss, aux_loss
